# baseline (speedup 1.0000x reference)
import functools

import jax
import jax.numpy as jnp
from jax import lax
from jax.experimental import pallas as pl
from jax.experimental.pallas import tpu as pltpu
from jax.experimental.pallas import tpu_sc as plsc

_NC = 2
_NS = 16
_NW = _NC * _NS
_LANES = 16
_VPAD = 1024


def _score_table_body(emb_ref, w_ref, out_ref, *, inv_len):
    out_ref[...] = jnp.zeros_like(out_ref)
    e = emb_ref[...]
    w = w_ref[...]
    out_ref[0 : e.shape[0], :] = jnp.sum(e * w, axis=1, keepdims=True) * inv_len


def _make_sc_pool(B, SW):
    R = B // _NW
    U = 4
    mesh = plsc.VectorSubcoreMesh(core_axis_name="c", subcore_axis_name="s")

    @functools.partial(
        pl.kernel,
        mesh=mesh,
        out_type=jax.ShapeDtypeStruct((B,), jnp.float32),
        scratch_types=[
            pltpu.VMEM((R, SW), jnp.int32),
            pltpu.VMEM((_VPAD,), jnp.float32),
            pltpu.VMEM((_LANES,), jnp.float32),
            pltpu.VMEM((R,), jnp.float32),
        ],
        compiler_params=pltpu.CompilerParams(needs_layout_passes=False),
    )
    def sc_pool(x_hbm, s_hbm, b_hbm, out_hbm, x_v, s_v, b_v, out_v):
        wid = lax.axis_index("s") * _NC + lax.axis_index("c")
        pltpu.sync_copy(s_hbm, s_v)
        pltpu.sync_copy(b_hbm, b_v)
        pltpu.sync_copy(x_hbm.at[pl.ds(wid * R, R)], x_v)
        lane = lax.iota(jnp.int32, _LANES)
        bv = b_v[...]

        def group(g, _):
            rows = lane + g * _LANES

            def step(t, carry):
                acc0, acc1, col = carry
                parts0 = []
                parts1 = []
                for u in range(U):
                    xw = plsc.load_gather(x_v, [rows, col + u])
                    loi = xw & 0xFFFF
                    hii = lax.shift_right_logical(xw, 16)
                    parts0.append(plsc.load_gather(s_v, [loi]))
                    parts1.append(plsc.load_gather(s_v, [hii]))
                acc0 = acc0 + ((parts0[0] + parts0[1])
                               + (parts0[2] + parts0[3]))
                acc1 = acc1 + ((parts1[0] + parts1[1])
                               + (parts1[2] + parts1[3]))
                return acc0, acc1, col + U

            zero = jnp.zeros((_LANES,), jnp.float32)
            col0 = jnp.zeros((_LANES,), jnp.int32)
            acc0, acc1, _ = lax.fori_loop(0, SW // U, step,
                                          (zero, zero, col0))
            z = acc0 + acc1 + bv
            out_v[pl.ds(g * _LANES, _LANES)] = 1.0 / (1.0 + jnp.exp(-z))
            return 0

        lax.fori_loop(0, R // _LANES, group, 0)
        pltpu.sync_copy(out_v, out_hbm.at[pl.ds(wid * R, R)])

    return sc_pool


def kernel(x, emb, W, b):
    B, S = x.shape
    V, D = emb.shape
    s2d = pl.pallas_call(
        functools.partial(_score_table_body, inv_len=1.0 / S),
        out_shape=jax.ShapeDtypeStruct((_VPAD, 1), jnp.float32),
    )(emb, W)
    s_flat = s2d.reshape(_VPAD)
    b16 = jnp.broadcast_to(b.astype(jnp.float32), (_LANES,))
    xi = x.astype(jnp.int32)
    xp = xi[:, : S // 2] | (xi[:, S // 2 :] << 16)
    out_flat = _make_sc_pool(B, S // 2)(xp, s_flat, b16)
    return out_flat.reshape(B, 1)


# speedup vs baseline: 1.1415x; 1.1415x over previous
import functools

import jax
import jax.numpy as jnp
from jax import lax
from jax.experimental import pallas as pl
from jax.experimental.pallas import tpu as pltpu
from jax.experimental.pallas import tpu_sc as plsc

_NC = 2
_NS = 16
_NW = _NC * _NS
_LANES = 16
_VPAD = 1024


def _pack_body(lo_ref, hi_ref, out_ref):
    out_ref[...] = lo_ref[...] | (hi_ref[...] << 16)


def _score_table_body(emb_ref, w_ref, out_ref, *, inv_len):
    out_ref[...] = jnp.zeros_like(out_ref)
    e = emb_ref[...]
    w = w_ref[...]
    out_ref[0 : e.shape[0], :] = jnp.sum(e * w, axis=1, keepdims=True) * inv_len


def _make_sc_pool(B, S):
    H = B // 2
    R = H // _NW
    C = 64
    NCH = R // C
    U = 4
    mesh = plsc.VectorSubcoreMesh(core_axis_name="c", subcore_axis_name="s")

    @functools.partial(
        pl.kernel,
        mesh=mesh,
        out_type=jax.ShapeDtypeStruct((B,), jnp.float32),
        scratch_types=[
            pltpu.VMEM((C, S), jnp.int32),
            pltpu.VMEM((C, S), jnp.int32),
            pltpu.VMEM((_VPAD,), jnp.float32),
            pltpu.VMEM((_LANES,), jnp.float32),
            pltpu.VMEM((2 * R,), jnp.float32),
            pltpu.SemaphoreType.DMA,
            pltpu.SemaphoreType.DMA,
        ],
        compiler_params=pltpu.CompilerParams(needs_layout_passes=False),
    )
    def sc_pool(x_hbm, s_hbm, b_hbm, out_hbm, x_v0, x_v1, s_v, b_v, out_v,
                sem0, sem1):
        wid = lax.axis_index("s") * _NC + lax.axis_index("c")
        base = wid * R
        bufs = (x_v0, x_v1)
        sems = (sem0, sem1)
        pending = {0: pltpu.async_copy(x_hbm.at[pl.ds(base, C)], x_v0, sem0)}
        pltpu.sync_copy(s_hbm, s_v)
        pltpu.sync_copy(b_hbm, b_v)
        lane = lax.iota(jnp.int32, _LANES)
        bv = b_v[...]

        for k in range(NCH):
            if k + 1 < NCH:
                pending[k + 1] = pltpu.async_copy(
                    x_hbm.at[pl.ds(base + (k + 1) * C, C)],
                    bufs[(k + 1) % 2], sems[(k + 1) % 2])
            pending.pop(k).wait()
            x_v = bufs[k % 2]

            def group(g, _, x_v=x_v, k=k):
                rows = lane + g * _LANES

                def step(t, carry):
                    acc0, acc1, col = carry
                    parts0 = []
                    parts1 = []
                    for u in range(U):
                        xw = plsc.load_gather(x_v, [rows, col + u])
                        loi = xw & 0xFFFF
                        hii = lax.shift_right_logical(xw, 16)
                        parts0.append(plsc.load_gather(s_v, [loi]))
                        parts1.append(plsc.load_gather(s_v, [hii]))
                    acc0 = acc0 + ((parts0[0] + parts0[1])
                                   + (parts0[2] + parts0[3]))
                    acc1 = acc1 + ((parts1[0] + parts1[1])
                                   + (parts1[2] + parts1[3]))
                    return acc0, acc1, col + U

                zero = jnp.zeros((_LANES,), jnp.float32)
                col0 = jnp.zeros((_LANES,), jnp.int32)
                acc0, acc1, _ = lax.fori_loop(0, S // U, step,
                                              (zero, zero, col0))
                off = k * C + g * _LANES
                out_v[pl.ds(off, _LANES)] = 1.0 / (1.0 + jnp.exp(-(acc0 + bv)))
                out_v[pl.ds(R + off, _LANES)] = (
                    1.0 / (1.0 + jnp.exp(-(acc1 + bv))))
                return 0

            lax.fori_loop(0, C // _LANES, group, 0)

        pltpu.sync_copy(out_v.at[pl.ds(0, R)], out_hbm.at[pl.ds(base, R)])
        pltpu.sync_copy(out_v.at[pl.ds(R, R)],
                        out_hbm.at[pl.ds(H + base, R)])

    return sc_pool


def kernel(x, emb, W, b):
    B, S = x.shape
    V, D = emb.shape
    s2d = pl.pallas_call(
        functools.partial(_score_table_body, inv_len=1.0 / S),
        out_shape=jax.ShapeDtypeStruct((_VPAD, 1), jnp.float32),
    )(emb, W)
    s_flat = s2d.reshape(_VPAD)
    b16 = jnp.broadcast_to(b.astype(jnp.float32), (_LANES,))
    BLK = 512
    H = B // 2
    xi = x.astype(jnp.int32)
    xp = pl.pallas_call(
        _pack_body,
        grid=(H // BLK,),
        in_specs=[pl.BlockSpec((BLK, S), lambda i: (i, 0)),
                  pl.BlockSpec((BLK, S), lambda i, n=H // BLK: (i + n, 0))],
        out_specs=pl.BlockSpec((BLK, S), lambda i: (i, 0)),
        out_shape=jax.ShapeDtypeStruct((H, S), jnp.int32),
    )(xi, xi)
    out_flat = _make_sc_pool(B, S)(xp, s_flat, b16)
    return out_flat.reshape(B, 1)


# speedup vs baseline: 1.1548x; 1.0117x over previous
import functools

import jax
import jax.numpy as jnp
from jax import lax
from jax.experimental import pallas as pl
from jax.experimental.pallas import tpu as pltpu
from jax.experimental.pallas import tpu_sc as plsc

_NC = 2
_NS = 16
_NW = _NC * _NS
_LANES = 16
_VPAD = 1024


def _pack_and_score_body(lo_ref, hi_ref, emb_ref, w_ref, out_ref, s_ref, *,
                         inv_len):
    out_ref[...] = lo_ref[...] | (hi_ref[...] << 16)

    @pl.when(pl.program_id(0) == 0)
    def _():
        s_ref[...] = jnp.zeros_like(s_ref)
        e = emb_ref[...]
        w = w_ref[...]
        s_ref[0 : e.shape[0], :] = (
            jnp.sum(e * w, axis=1, keepdims=True) * inv_len)


def _make_sc_pool(B, S):
    H = B // 2
    R = H // _NW
    C = 128
    NCH = R // C
    U = 8
    mesh = plsc.VectorSubcoreMesh(core_axis_name="c", subcore_axis_name="s")

    @functools.partial(
        pl.kernel,
        mesh=mesh,
        out_type=jax.ShapeDtypeStruct((B,), jnp.float32),
        scratch_types=[
            pltpu.VMEM((C, S), jnp.int32),
            pltpu.VMEM((C, S), jnp.int32),
            pltpu.VMEM((_VPAD,), jnp.float32),
            pltpu.VMEM((_LANES,), jnp.float32),
            pltpu.VMEM((2 * R,), jnp.float32),
            pltpu.SemaphoreType.DMA,
            pltpu.SemaphoreType.DMA,
        ],
        compiler_params=pltpu.CompilerParams(needs_layout_passes=False),
    )
    def sc_pool(x_hbm, s_hbm, b_hbm, out_hbm, x_v0, x_v1, s_v, b_v, out_v,
                sem0, sem1):
        wid = lax.axis_index("s") * _NC + lax.axis_index("c")
        base = wid * R
        bufs = (x_v0, x_v1)
        sems = (sem0, sem1)
        pending = {0: pltpu.async_copy(x_hbm.at[pl.ds(base, C)], x_v0, sem0)}
        pltpu.sync_copy(s_hbm, s_v)
        pltpu.sync_copy(b_hbm, b_v)
        lane = lax.iota(jnp.int32, _LANES)
        bv = b_v[...]

        for k in range(NCH):
            if k + 1 < NCH:
                pending[k + 1] = pltpu.async_copy(
                    x_hbm.at[pl.ds(base + (k + 1) * C, C)],
                    bufs[(k + 1) % 2], sems[(k + 1) % 2])
            pending.pop(k).wait()
            x_v = bufs[k % 2]

            def group(g, _, x_v=x_v, k=k):
                rows = lane + g * _LANES

                def step(t, carry):
                    acc0, acc1, col = carry
                    parts0 = []
                    parts1 = []
                    for u in range(U):
                        xw = plsc.load_gather(x_v, [rows, col + u])
                        loi = xw & 0xFFFF
                        hii = lax.shift_right_logical(xw, 16)
                        parts0.append(plsc.load_gather(s_v, [loi]))
                        parts1.append(plsc.load_gather(s_v, [hii]))

                    def tree(ps):
                        while len(ps) > 1:
                            ps = [a + c for a, c in zip(ps[::2], ps[1::2])]
                        return ps[0]

                    acc0 = acc0 + tree(parts0)
                    acc1 = acc1 + tree(parts1)
                    return acc0, acc1, col + U

                zero = jnp.zeros((_LANES,), jnp.float32)
                col0 = jnp.zeros((_LANES,), jnp.int32)
                acc0, acc1, _ = lax.fori_loop(0, S // U, step,
                                              (zero, zero, col0))
                off = k * C + g * _LANES
                out_v[pl.ds(off, _LANES)] = 1.0 / (1.0 + jnp.exp(-(acc0 + bv)))
                out_v[pl.ds(R + off, _LANES)] = (
                    1.0 / (1.0 + jnp.exp(-(acc1 + bv))))
                return 0

            lax.fori_loop(0, C // _LANES, group, 0)

        pltpu.sync_copy(out_v.at[pl.ds(0, R)], out_hbm.at[pl.ds(base, R)])
        pltpu.sync_copy(out_v.at[pl.ds(R, R)],
                        out_hbm.at[pl.ds(H + base, R)])

    return sc_pool


def kernel(x, emb, W, b):
    B, S = x.shape
    V, D = emb.shape
    b16 = jnp.broadcast_to(b.astype(jnp.float32), (_LANES,))
    BLK = 512
    H = B // 2
    xi = x.astype(jnp.int32)
    xp, s2d = pl.pallas_call(
        functools.partial(_pack_and_score_body, inv_len=1.0 / S),
        grid=(H // BLK,),
        in_specs=[
            pl.BlockSpec((BLK, S), lambda i: (i, 0)),
            pl.BlockSpec((BLK, S), lambda i, n=H // BLK: (i + n, 0)),
            pl.BlockSpec((V, D), lambda i: (0, 0)),
            pl.BlockSpec((1, D), lambda i: (0, 0)),
        ],
        out_specs=[pl.BlockSpec((BLK, S), lambda i: (i, 0)),
                   pl.BlockSpec((_VPAD, 1), lambda i: (0, 0))],
        out_shape=[jax.ShapeDtypeStruct((H, S), jnp.int32),
                   jax.ShapeDtypeStruct((_VPAD, 1), jnp.float32)],
    )(xi, xi, emb, W)
    s_flat = s2d.reshape(_VPAD)
    out_flat = _make_sc_pool(B, S)(xp, s_flat, b16)
    return out_flat.reshape(B, 1)


# speedup vs baseline: 1.4491x; 1.2548x over previous
import functools

import jax
import jax.numpy as jnp
from jax import lax
from jax.experimental import pallas as pl
from jax.experimental.pallas import tpu as pltpu
from jax.experimental.pallas import tpu_sc as plsc

_NC = 2
_NS = 16
_NW = _NC * _NS
_LANES = 16
_VPAD = 1024


def _pack_and_score_body(lo_ref, hi_ref, emb_ref, w_ref, out_ref, s_ref, *,
                         inv_len):
    out_ref[...] = lo_ref[...] | (hi_ref[...] << 16)

    @pl.when(pl.program_id(0) == 0)
    def _():
        s_ref[...] = jnp.zeros_like(s_ref)
        e = emb_ref[...]
        w = w_ref[...]
        s_ref[0 : e.shape[0], :] = (
            jnp.sum(e * w, axis=1, keepdims=True) * inv_len)


def _make_sc_pool(B, S):
    H = B // 2
    R = H // _NW
    C = 128
    NCH = R // C
    U = 8
    mesh = plsc.VectorSubcoreMesh(core_axis_name="c", subcore_axis_name="s")

    @functools.partial(
        pl.kernel,
        mesh=mesh,
        out_type=jax.ShapeDtypeStruct((B,), jnp.float32),
        scratch_types=[
            pltpu.VMEM((C, S), jnp.int32),
            pltpu.VMEM((C, S), jnp.int32),
            pltpu.VMEM((_VPAD,), jnp.float32),
            pltpu.VMEM((_LANES,), jnp.float32),
            pltpu.VMEM((2 * R,), jnp.float32),
            pltpu.SemaphoreType.DMA,
            pltpu.SemaphoreType.DMA,
        ],
        compiler_params=pltpu.CompilerParams(needs_layout_passes=False),
    )
    def sc_pool(x_hbm, s_hbm, b_hbm, out_hbm, x_v0, x_v1, s_v, b_v, out_v,
                sem0, sem1):
        wid = lax.axis_index("s") * _NC + lax.axis_index("c")
        base = wid * R
        bufs = (x_v0, x_v1)
        sems = (sem0, sem1)
        pending = {0: pltpu.async_copy(x_hbm.at[pl.ds(base, C)], x_v0, sem0)}
        pltpu.sync_copy(s_hbm, s_v)
        pltpu.sync_copy(b_hbm, b_v)
        lane = lax.iota(jnp.int32, _LANES)
        bv = b_v[...]

        for k in range(NCH):
            if k + 1 < NCH:
                pending[k + 1] = pltpu.async_copy(
                    x_hbm.at[pl.ds(base + (k + 1) * C, C)],
                    bufs[(k + 1) % 2], sems[(k + 1) % 2])
            pending.pop(k).wait()
            x_v = bufs[k % 2]

            def group(g, _, x_v=x_v, k=k):
                rows = lane + g * _LANES

                def step(t, carry):
                    acc0, acc1, col = carry
                    parts0 = []
                    parts1 = []
                    for u in range(U):
                        xw = plsc.load_gather(x_v, [rows, col + u])
                        loi = xw & 0xFFFF
                        hii = lax.shift_right_logical(xw, 16)
                        parts0.append(plsc.load_gather(s_v, [loi]))
                        parts1.append(plsc.load_gather(s_v, [hii]))

                    def tree(ps):
                        while len(ps) > 1:
                            ps = [a + c for a, c in zip(ps[::2], ps[1::2])]
                        return ps[0]

                    acc0 = acc0 + tree(parts0)
                    acc1 = acc1 + tree(parts1)
                    col = col + U
                    col = jnp.where(col >= S, col - S, col)
                    return acc0, acc1, col

                zero = jnp.zeros((_LANES,), jnp.float32)
                col0 = lane * U
                acc0, acc1, _ = lax.fori_loop(0, S // U, step,
                                              (zero, zero, col0))
                off = k * C + g * _LANES
                out_v[pl.ds(off, _LANES)] = 1.0 / (1.0 + jnp.exp(-(acc0 + bv)))
                out_v[pl.ds(R + off, _LANES)] = (
                    1.0 / (1.0 + jnp.exp(-(acc1 + bv))))
                return 0

            lax.fori_loop(0, C // _LANES, group, 0)

        pltpu.sync_copy(out_v.at[pl.ds(0, R)], out_hbm.at[pl.ds(base, R)])
        pltpu.sync_copy(out_v.at[pl.ds(R, R)],
                        out_hbm.at[pl.ds(H + base, R)])

    return sc_pool


def kernel(x, emb, W, b):
    B, S = x.shape
    V, D = emb.shape
    b16 = jnp.broadcast_to(b.astype(jnp.float32), (_LANES,))
    BLK = 512
    H = B // 2
    xi = x.astype(jnp.int32)
    xp, s2d = pl.pallas_call(
        functools.partial(_pack_and_score_body, inv_len=1.0 / S),
        grid=(H // BLK,),
        in_specs=[
            pl.BlockSpec((BLK, S), lambda i: (i, 0)),
            pl.BlockSpec((BLK, S), lambda i, n=H // BLK: (i + n, 0)),
            pl.BlockSpec((V, D), lambda i: (0, 0)),
            pl.BlockSpec((1, D), lambda i: (0, 0)),
        ],
        out_specs=[pl.BlockSpec((BLK, S), lambda i: (i, 0)),
                   pl.BlockSpec((_VPAD, 1), lambda i: (0, 0))],
        out_shape=[jax.ShapeDtypeStruct((H, S), jnp.int32),
                   jax.ShapeDtypeStruct((_VPAD, 1), jnp.float32)],
    )(xi, xi, emb, W)
    s_flat = s2d.reshape(_VPAD)
    out_flat = _make_sc_pool(B, S)(xp, s_flat, b16)
    return out_flat.reshape(B, 1)


# speedup vs baseline: 1.4616x; 1.0087x over previous
import functools

import jax
import jax.numpy as jnp
from jax import lax
from jax.experimental import pallas as pl
from jax.experimental.pallas import tpu as pltpu
from jax.experimental.pallas import tpu_sc as plsc

_NC = 2
_NS = 16
_NW = _NC * _NS
_LANES = 16
_VPAD = 1024


def _pack_and_score_body(lo_ref, hi_ref, emb_ref, w_ref, out_ref, s_ref, *,
                         inv_len):
    out_ref[...] = lo_ref[...] | (hi_ref[...] << 16)

    @pl.when(pl.program_id(0) == 0)
    def _():
        s_ref[...] = jnp.zeros_like(s_ref)
        e = emb_ref[...]
        w = w_ref[...]
        s_ref[0 : e.shape[0], :] = (
            jnp.sum(e * w, axis=1, keepdims=True) * inv_len)


def _make_sc_pool(B, S):
    H = B // 2
    R = H // _NW
    C = 64
    NCH = R // C
    U = 8
    mesh = plsc.VectorSubcoreMesh(core_axis_name="c", subcore_axis_name="s")

    @functools.partial(
        pl.kernel,
        mesh=mesh,
        out_type=jax.ShapeDtypeStruct((B,), jnp.float32),
        scratch_types=[
            pltpu.VMEM((C, S), jnp.int32),
            pltpu.VMEM((C, S), jnp.int32),
            pltpu.VMEM((_VPAD,), jnp.float32),
            pltpu.VMEM((_LANES,), jnp.float32),
            pltpu.VMEM((2 * R,), jnp.float32),
            pltpu.SemaphoreType.DMA,
            pltpu.SemaphoreType.DMA,
        ],
        compiler_params=pltpu.CompilerParams(needs_layout_passes=False),
    )
    def sc_pool(x_hbm, s_hbm, b_hbm, out_hbm, x_v0, x_v1, s_v, b_v, out_v,
                sem0, sem1):
        wid = lax.axis_index("s") * _NC + lax.axis_index("c")
        base = wid * R
        bufs = (x_v0, x_v1)
        sems = (sem0, sem1)
        pending = {0: pltpu.async_copy(x_hbm.at[pl.ds(base, C)], x_v0, sem0)}
        pltpu.sync_copy(s_hbm, s_v)
        pltpu.sync_copy(b_hbm, b_v)
        lane = lax.iota(jnp.int32, _LANES)
        bv = b_v[...]

        for k in range(NCH):
            if k + 1 < NCH:
                pending[k + 1] = pltpu.async_copy(
                    x_hbm.at[pl.ds(base + (k + 1) * C, C)],
                    bufs[(k + 1) % 2], sems[(k + 1) % 2])
            pending.pop(k).wait()
            x_v = bufs[k % 2]

            def group(g, _, x_v=x_v, k=k):
                rows = lane + g * _LANES

                def step(t, carry):
                    acc0, acc1, col = carry
                    parts0 = []
                    parts1 = []
                    for u in range(U):
                        xw = plsc.load_gather(x_v, [rows, col + u])
                        loi = xw & 0xFFFF
                        hii = lax.shift_right_logical(xw, 16)
                        parts0.append(plsc.load_gather(s_v, [loi]))
                        parts1.append(plsc.load_gather(s_v, [hii]))

                    def tree(ps):
                        while len(ps) > 1:
                            ps = [a + c for a, c in zip(ps[::2], ps[1::2])]
                        return ps[0]

                    acc0 = acc0 + tree(parts0)
                    acc1 = acc1 + tree(parts1)
                    col = col + U
                    col = jnp.where(col >= S, col - S, col)
                    return acc0, acc1, col

                zero = jnp.zeros((_LANES,), jnp.float32)
                col0 = lane * U
                acc0, acc1, _ = lax.fori_loop(0, S // U, step,
                                              (zero, zero, col0))
                off = k * C + g * _LANES
                out_v[pl.ds(off, _LANES)] = 1.0 / (1.0 + jnp.exp(-(acc0 + bv)))
                out_v[pl.ds(R + off, _LANES)] = (
                    1.0 / (1.0 + jnp.exp(-(acc1 + bv))))
                return 0

            lax.fori_loop(0, C // _LANES, group, 0)

        pltpu.sync_copy(out_v.at[pl.ds(0, R)], out_hbm.at[pl.ds(base, R)])
        pltpu.sync_copy(out_v.at[pl.ds(R, R)],
                        out_hbm.at[pl.ds(H + base, R)])

    return sc_pool


def kernel(x, emb, W, b):
    B, S = x.shape
    V, D = emb.shape
    b16 = jnp.broadcast_to(b.astype(jnp.float32), (_LANES,))
    BLK = 512
    H = B // 2
    xi = x.astype(jnp.int32)
    xp, s2d = pl.pallas_call(
        functools.partial(_pack_and_score_body, inv_len=1.0 / S),
        grid=(H // BLK,),
        in_specs=[
            pl.BlockSpec((BLK, S), lambda i: (i, 0)),
            pl.BlockSpec((BLK, S), lambda i, n=H // BLK: (i + n, 0)),
            pl.BlockSpec((V, D), lambda i: (0, 0)),
            pl.BlockSpec((1, D), lambda i: (0, 0)),
        ],
        out_specs=[pl.BlockSpec((BLK, S), lambda i: (i, 0)),
                   pl.BlockSpec((_VPAD, 1), lambda i: (0, 0))],
        out_shape=[jax.ShapeDtypeStruct((H, S), jnp.int32),
                   jax.ShapeDtypeStruct((_VPAD, 1), jnp.float32)],
    )(xi, xi, emb, W)
    s_flat = s2d.reshape(_VPAD)
    out_flat = _make_sc_pool(B, S)(xp, s_flat, b16)
    return out_flat.reshape(B, 1)
